# Initial kernel scaffold; baseline (speedup 1.0000x reference)
import jax
import jax.numpy as jnp
from jax.experimental import pallas as pl


def kernel(x, edge_index, W1, b1, g1, be1, W2, b2, g2, be2, W3, b3, g3, be3, Wh, bh):
    raise NotImplementedError("write your pallas kernel here")


# speedup vs baseline: 17.1685x; 17.1685x over previous
import functools

import jax
import jax.numpy as jnp
from jax import lax
from jax.experimental import pallas as pl
from jax.experimental.pallas import tpu as pltpu
from jax.experimental.pallas import tpu_sc as plsc

_N = 10000
_E = 320000
_D = 128
_NC = 2
_NS = 16
_NW = _NC * _NS
_CHUNK = 128
_ROWS = 79
_EP = _NW * _ROWS * _CHUNK
_PAD = 240
_NP = _N + _PAD
_RPT = _NP // _NS
_DW = 128


@functools.lru_cache(maxsize=None)
def _build_deg_kernel():
    mesh = plsc.VectorSubcoreMesh(core_axis_name="c", subcore_axis_name="s")

    @functools.partial(
        pl.kernel,
        mesh=mesh,
        out_type=jax.ShapeDtypeStruct((_NC, _NP, _DW), jnp.float32),
        scratch_types=[
            pltpu.VMEM((_ROWS, _CHUNK), jnp.int32),
            pltpu.VMEM((_CHUNK, _DW), jnp.float32),
            pltpu.VMEM_SHARED((_NP, _DW), jnp.float32),
        ],
    )
    def deg_kernel(dst_hbm, zeros_hbm, ones_hbm, out_hbm, dst_v, ones_v,
                   acc_sh):
        c = lax.axis_index("c")
        s = lax.axis_index("s")
        wid = s * _NC + c
        pltpu.sync_copy(zeros_hbm, acc_sh.at[pl.ds(s * _RPT, _RPT)])
        pltpu.sync_copy(dst_hbm.at[wid], dst_v)
        pltpu.sync_copy(ones_hbm, ones_v)
        plsc.subcore_barrier()

        def body(j, carry):
            pltpu.sync_copy(ones_v, acc_sh.at[dst_v.at[j]], add=True)
            return carry

        lax.fori_loop(0, _ROWS, body, 0)
        plsc.subcore_barrier()
        pltpu.sync_copy(acc_sh.at[pl.ds(s * _RPT, _RPT)],
                        out_hbm.at[c, pl.ds(s * _RPT, _RPT)])

    return deg_kernel


@functools.lru_cache(maxsize=None)
def _build_agg_kernel():
    mesh = plsc.VectorSubcoreMesh(core_axis_name="c", subcore_axis_name="s")

    @functools.partial(
        pl.kernel,
        mesh=mesh,
        out_type=jax.ShapeDtypeStruct((_NC, _NP, _D), jnp.float32),
        scratch_types=[
            pltpu.VMEM((_ROWS, _CHUNK), jnp.int32),
            pltpu.VMEM((_ROWS, _CHUNK), jnp.int32),
            pltpu.VMEM((_CHUNK, _D), jnp.float32),
            pltpu.VMEM_SHARED((_NP, _D), jnp.float32),
            pltpu.SemaphoreType.DMA,
        ],
    )
    def agg_kernel(h_hbm, src_hbm, dst_hbm, zeros_hbm, out_hbm,
                   src_v, dst_v, rows_v, acc_sh, sem):
        c = lax.axis_index("c")
        s = lax.axis_index("s")
        wid = s * _NC + c
        pltpu.sync_copy(zeros_hbm, acc_sh.at[pl.ds(s * _RPT, _RPT)])
        pltpu.sync_copy(src_hbm.at[wid], src_v)
        pltpu.sync_copy(dst_hbm.at[wid], dst_v)
        plsc.subcore_barrier()

        def body(j, carry):
            pltpu.async_copy(h_hbm.at[src_v.at[j]], rows_v, sem).wait()
            pltpu.sync_copy(rows_v, acc_sh.at[dst_v.at[j]], add=True)
            return carry

        lax.fori_loop(0, _ROWS, body, 0)
        plsc.subcore_barrier()
        pltpu.sync_copy(acc_sh.at[pl.ds(s * _RPT, _RPT)],
                        out_hbm.at[c, pl.ds(s * _RPT, _RPT)])

    return agg_kernel


def _deg_call(dst_p, zeros_deg, ones_deg):
    return _build_deg_kernel()(dst_p, zeros_deg, ones_deg)


def _agg_call(hp, src_p, dst_p, zeros_rows):
    return _build_agg_kernel()(hp, src_p, dst_p, zeros_rows)


def _dinv_col(deg_ref):
    d = deg_ref[0, :, 0:1] + deg_ref[1, :, 0:1] + 1.0
    return lax.rsqrt(d)


def _matmul_t(a, w_ref):
    return lax.dot_general(a, w_ref[...], (((1,), (1,)), ((), ())),
                           preferred_element_type=jnp.float32,
                           precision=lax.Precision.HIGHEST)


def _ln_gelu(t, g_ref, be_ref):
    m = jnp.mean(t, axis=-1, keepdims=True)
    tc = t - m
    v = jnp.mean(tc * tc, axis=-1, keepdims=True)
    ln = tc * lax.rsqrt(v + 1e-5) * g_ref[...] + be_ref[...]
    return ln * 0.5 * (1.0 + lax.erf(ln * (2.0 ** -0.5)))


def _tc_in_body(x_ref, w_ref, deg_ref, o_ref):
    o_ref[...] = _dinv_col(deg_ref) * _matmul_t(x_ref[...], w_ref)


def _tc_mid_body(agg_ref, hp_ref, deg_ref, b_ref, g_ref, be_ref, w_ref, o_ref):
    dinv = _dinv_col(deg_ref)
    t = dinv * (agg_ref[0] + agg_ref[1] + hp_ref[...]) + b_ref[...]
    act = _ln_gelu(t, g_ref, be_ref)
    o_ref[...] = dinv * _matmul_t(act, w_ref)


def _tc_out_body(agg_ref, hp_ref, deg_ref, b_ref, g_ref, be_ref,
                 wh_ref, bh_ref, o_ref):
    dinv = _dinv_col(deg_ref)
    t = dinv * (agg_ref[0] + agg_ref[1] + hp_ref[...]) + b_ref[...]
    act = _ln_gelu(t, g_ref, be_ref)
    y = jnp.sum(act * wh_ref[...], axis=-1, keepdims=True) + bh_ref[...]
    o_ref[...] = y[:_N]


def _tc_in(x_p, W1, deg):
    return pl.pallas_call(
        _tc_in_body,
        out_shape=jax.ShapeDtypeStruct((_NP, _D), jnp.float32),
    )(x_p, W1, deg)


def _tc_mid(agg, hp, deg, b, g, be, Wn):
    return pl.pallas_call(
        _tc_mid_body,
        out_shape=jax.ShapeDtypeStruct((_NP, _D), jnp.float32),
    )(agg, hp, deg, b.reshape(1, _D), g.reshape(1, _D), be.reshape(1, _D), Wn)


def _tc_out(agg, hp, deg, b, g, be, Wh, bh):
    return pl.pallas_call(
        _tc_out_body,
        out_shape=jax.ShapeDtypeStruct((_N, 1), jnp.float32),
    )(agg, hp, deg, b.reshape(1, _D), g.reshape(1, _D), be.reshape(1, _D),
      Wh, bh.reshape(1, 1))


def kernel(x, edge_index, W1, b1, g1, be1, W2, b2, g2, be2, W3, b3, g3, be3,
           Wh, bh):
    src = edge_index[0]
    dst = edge_index[1]
    npad = _EP - _E
    pad_idx = _N + (jnp.arange(npad, dtype=jnp.int32) % _PAD)
    src_p = jnp.concatenate([src, pad_idx]).reshape(_NW, _ROWS, _CHUNK)
    dst_p = jnp.concatenate([dst, pad_idx]).reshape(_NW, _ROWS, _CHUNK)
    x_p = jnp.pad(x, ((0, _NP - _N), (0, 0)))

    zeros_rows = jnp.zeros((_RPT, _D), jnp.float32)
    zeros_deg = jnp.zeros((_RPT, _DW), jnp.float32)
    ones_deg = jnp.zeros((_CHUNK, _DW), jnp.float32).at[:, 0].set(1.0)

    deg = _deg_call(dst_p, zeros_deg, ones_deg)
    hp1 = _tc_in(x_p, W1, deg)
    agg1 = _agg_call(hp1, src_p, dst_p, zeros_rows)
    hp2 = _tc_mid(agg1, hp1, deg, b1, g1, be1, W2)
    agg2 = _agg_call(hp2, src_p, dst_p, zeros_rows)
    hp3 = _tc_mid(agg2, hp2, deg, b2, g2, be2, W3)
    agg3 = _agg_call(hp3, src_p, dst_p, zeros_rows)
    y = _tc_out(agg3, hp3, deg, b3, g3, be3, Wh, bh)
    return y[:, 0]


# speedup vs baseline: 21.1886x; 1.2342x over previous
import functools

import jax
import jax.numpy as jnp
from jax import lax
from jax.experimental import pallas as pl
from jax.experimental.pallas import tpu as pltpu
from jax.experimental.pallas import tpu_sc as plsc

_N = 10000
_E = 320000
_D = 128
_NC = 2
_NS = 16
_NW = _NC * _NS
_CHUNK = 128
_HC = 64
_ROWS = 80
_EP = _NW * _ROWS * _CHUNK
_PAD = 240
_NP = _N + _PAD
_RPT = _NP // _NS
_DW = 128


@functools.lru_cache(maxsize=None)
def _build_deg_kernel():
    mesh = plsc.VectorSubcoreMesh(core_axis_name="c", subcore_axis_name="s")

    @functools.partial(
        pl.kernel,
        mesh=mesh,
        out_type=jax.ShapeDtypeStruct((_NC, _NP, _DW), jnp.float32),
        scratch_types=[
            pltpu.VMEM((_ROWS, _CHUNK), jnp.int32),
            pltpu.VMEM((_CHUNK, _DW), jnp.float32),
            pltpu.VMEM_SHARED((_NP, _DW), jnp.float32),
        ],
    )
    def deg_kernel(dst_hbm, zeros_hbm, ones_hbm, out_hbm, dst_v, ones_v,
                   acc_sh):
        c = lax.axis_index("c")
        s = lax.axis_index("s")
        wid = s * _NC + c
        pltpu.sync_copy(zeros_hbm, acc_sh.at[pl.ds(s * _RPT, _RPT)])
        pltpu.sync_copy(dst_hbm.at[wid], dst_v)
        pltpu.sync_copy(ones_hbm, ones_v)
        plsc.subcore_barrier()

        def body(j, carry):
            pltpu.sync_copy(ones_v, acc_sh.at[dst_v.at[j]], add=True)
            return carry

        lax.fori_loop(0, _ROWS, body, 0)
        plsc.subcore_barrier()
        pltpu.sync_copy(acc_sh.at[pl.ds(s * _RPT, _RPT)],
                        out_hbm.at[c, pl.ds(s * _RPT, _RPT)])

    return deg_kernel


@functools.lru_cache(maxsize=None)
def _build_agg_kernel():
    mesh = plsc.VectorSubcoreMesh(core_axis_name="c", subcore_axis_name="s")

    @functools.partial(
        pl.kernel,
        mesh=mesh,
        out_type=jax.ShapeDtypeStruct((_NC, _NP, _D), jnp.float32),
        scratch_types=[
            pltpu.VMEM((_ROWS, _CHUNK), jnp.int32),
            pltpu.VMEM((_ROWS, _CHUNK), jnp.int32),
            pltpu.VMEM((_HC, _D), jnp.float32),
            pltpu.VMEM((_HC, _D), jnp.float32),
            pltpu.VMEM_SHARED((_NP, _D), jnp.float32),
            pltpu.SemaphoreType.DMA,
            pltpu.SemaphoreType.DMA,
        ],
    )
    def agg_kernel(h_hbm, src_hbm, dst_hbm, zeros_hbm, out_hbm,
                   src_v, dst_v, rows_a, rows_b, acc_sh, sem_a, sem_b):
        c = lax.axis_index("c")
        s = lax.axis_index("s")
        wid = s * _NC + c
        pltpu.sync_copy(zeros_hbm, acc_sh.at[pl.ds(s * _RPT, _RPT)])
        pltpu.sync_copy(src_hbm.at[wid], src_v)
        pltpu.sync_copy(dst_hbm.at[wid], dst_v)
        plsc.subcore_barrier()

        pltpu.async_copy(h_hbm.at[src_v.at[0, pl.ds(0, _HC)]], rows_a, sem_a)

        def body(i, carry):
            pltpu.async_copy(h_hbm.at[src_v.at[i, pl.ds(_HC, _HC)]],
                             rows_b, sem_b)
            pltpu.make_async_copy(h_hbm.at[src_v.at[i, pl.ds(0, _HC)]],
                                  rows_a, sem_a).wait()
            pltpu.sync_copy(rows_a, acc_sh.at[dst_v.at[i, pl.ds(0, _HC)]],
                            add=True)

            @pl.when(i < _ROWS - 1)
            def _():
                pltpu.async_copy(h_hbm.at[src_v.at[i + 1, pl.ds(0, _HC)]],
                                 rows_a, sem_a)

            pltpu.make_async_copy(h_hbm.at[src_v.at[i, pl.ds(_HC, _HC)]],
                                  rows_b, sem_b).wait()
            pltpu.sync_copy(rows_b, acc_sh.at[dst_v.at[i, pl.ds(_HC, _HC)]],
                            add=True)
            return carry

        lax.fori_loop(0, _ROWS, body, 0)
        plsc.subcore_barrier()
        pltpu.sync_copy(acc_sh.at[pl.ds(s * _RPT, _RPT)],
                        out_hbm.at[c, pl.ds(s * _RPT, _RPT)])

    return agg_kernel


def _deg_call(dst_p, zeros_deg, ones_deg):
    return _build_deg_kernel()(dst_p, zeros_deg, ones_deg)


def _agg_call(hp, src_p, dst_p, zeros_rows):
    return _build_agg_kernel()(hp, src_p, dst_p, zeros_rows)


def _dinv_col(deg_ref):
    d = deg_ref[0, :, 0:1] + deg_ref[1, :, 0:1] + 1.0
    return lax.rsqrt(d)


def _matmul_t(a, w_ref):
    return lax.dot_general(a, w_ref[...], (((1,), (1,)), ((), ())),
                           preferred_element_type=jnp.float32,
                           precision=lax.Precision.HIGHEST)


def _ln_gelu(t, g_ref, be_ref):
    m = jnp.mean(t, axis=-1, keepdims=True)
    tc = t - m
    v = jnp.mean(tc * tc, axis=-1, keepdims=True)
    ln = tc * lax.rsqrt(v + 1e-5) * g_ref[...] + be_ref[...]
    return ln * 0.5 * (1.0 + lax.erf(ln * (2.0 ** -0.5)))


def _tc_in_body(x_ref, w_ref, deg_ref, o_ref):
    o_ref[...] = _dinv_col(deg_ref) * _matmul_t(x_ref[...], w_ref)


def _tc_mid_body(agg_ref, hp_ref, deg_ref, b_ref, g_ref, be_ref, w_ref, o_ref):
    dinv = _dinv_col(deg_ref)
    t = dinv * (agg_ref[0] + agg_ref[1] + hp_ref[...]) + b_ref[...]
    act = _ln_gelu(t, g_ref, be_ref)
    o_ref[...] = dinv * _matmul_t(act, w_ref)


def _tc_out_body(agg_ref, hp_ref, deg_ref, b_ref, g_ref, be_ref,
                 wh_ref, bh_ref, o_ref):
    dinv = _dinv_col(deg_ref)
    t = dinv * (agg_ref[0] + agg_ref[1] + hp_ref[...]) + b_ref[...]
    act = _ln_gelu(t, g_ref, be_ref)
    y = jnp.sum(act * wh_ref[...], axis=-1, keepdims=True) + bh_ref[...]
    o_ref[...] = y[:_N]


def _tc_in(x_p, W1, deg):
    return pl.pallas_call(
        _tc_in_body,
        out_shape=jax.ShapeDtypeStruct((_NP, _D), jnp.float32),
    )(x_p, W1, deg)


def _tc_mid(agg, hp, deg, b, g, be, Wn):
    return pl.pallas_call(
        _tc_mid_body,
        out_shape=jax.ShapeDtypeStruct((_NP, _D), jnp.float32),
    )(agg, hp, deg, b.reshape(1, _D), g.reshape(1, _D), be.reshape(1, _D), Wn)


def _tc_out(agg, hp, deg, b, g, be, Wh, bh):
    return pl.pallas_call(
        _tc_out_body,
        out_shape=jax.ShapeDtypeStruct((_N, 1), jnp.float32),
    )(agg, hp, deg, b.reshape(1, _D), g.reshape(1, _D), be.reshape(1, _D),
      Wh, bh.reshape(1, 1))


def kernel(x, edge_index, W1, b1, g1, be1, W2, b2, g2, be2, W3, b3, g3, be3,
           Wh, bh):
    src = edge_index[0]
    dst = edge_index[1]
    npad = _EP - _E
    pad_idx = _N + (jnp.arange(npad, dtype=jnp.int32) % _PAD)
    src_p = jnp.concatenate([src, pad_idx]).reshape(_NW, _ROWS, _CHUNK)
    dst_p = jnp.concatenate([dst, pad_idx]).reshape(_NW, _ROWS, _CHUNK)
    x_p = jnp.pad(x, ((0, _NP - _N), (0, 0)))

    zeros_rows = jnp.zeros((_RPT, _D), jnp.float32)
    zeros_deg = jnp.zeros((_RPT, _DW), jnp.float32)
    ones_deg = jnp.zeros((_CHUNK, _DW), jnp.float32).at[:, 0].set(1.0)

    deg = _deg_call(dst_p, zeros_deg, ones_deg)
    hp1 = _tc_in(x_p, W1, deg)
    agg1 = _agg_call(hp1, src_p, dst_p, zeros_rows)
    hp2 = _tc_mid(agg1, hp1, deg, b1, g1, be1, W2)
    agg2 = _agg_call(hp2, src_p, dst_p, zeros_rows)
    hp3 = _tc_mid(agg2, hp2, deg, b2, g2, be2, W3)
    agg3 = _agg_call(hp3, src_p, dst_p, zeros_rows)
    y = _tc_out(agg3, hp3, deg, b3, g3, be3, Wh, bh)
    return y[:, 0]


# speedup vs baseline: 21.4743x; 1.0135x over previous
import functools

import jax
import jax.numpy as jnp
from jax import lax
from jax.experimental import pallas as pl
from jax.experimental.pallas import tpu as pltpu
from jax.experimental.pallas import tpu_sc as plsc

_N = 10000
_E = 320000
_D = 128
_NC = 2
_NS = 16
_NW = _NC * _NS
_CHUNK = 128
_HC = 64
_SC4 = 4
_ESC = _E // (_CHUNK * _SC4)
_SPW = _ESC // _NW
_XS = _ESC - _SPW * _NW
_RA = 632
_RB = _N - (_NS - 1) * _RA


def _worker_chunks(c, s):
    wid = s * _NC + c
    base = wid * _SPW + jnp.minimum(wid, _XS)
    nj = (_SPW + jnp.where(wid < _XS, 1, 0)) * _SC4
    return wid, base, nj


def _stage_idx(e_hbm, plane, wid, base, idx_v):
    @pl.when(wid < _XS)
    def _():
        pltpu.sync_copy(e_hbm.at[plane, pl.ds(base, _SPW + 1)], idx_v)

    @pl.when(wid >= _XS)
    def _():
        pltpu.sync_copy(e_hbm.at[plane, pl.ds(base, _SPW)],
                        idx_v.at[pl.ds(0, _SPW)])


def _copy_acc(src_fn, dst_fn, s):
    @pl.when(s < _NS - 1)
    def _():
        pltpu.sync_copy(src_fn(_RA, s * _RA), dst_fn(_RA, s * _RA))

    @pl.when(s == _NS - 1)
    def _():
        pltpu.sync_copy(src_fn(_RB, s * _RA), dst_fn(_RB, s * _RA))


@functools.lru_cache(maxsize=None)
def _build_deg_kernel():
    mesh = plsc.VectorSubcoreMesh(core_axis_name="c", subcore_axis_name="s")

    @functools.partial(
        pl.kernel,
        mesh=mesh,
        out_type=jax.ShapeDtypeStruct((_NC, _N, _D), jnp.float32),
        scratch_types=[
            pltpu.VMEM((_SPW + 1, _SC4, _CHUNK), jnp.int32),
            pltpu.VMEM((_CHUNK, _D), jnp.float32),
            pltpu.VMEM_SHARED((_N, _D), jnp.float32),
        ],
    )
    def deg_kernel(e_hbm, zeros_hbm, ones_hbm, out_hbm, dst_v, ones_v,
                   acc_sh):
        c = lax.axis_index("c")
        s = lax.axis_index("s")
        wid, base, nj = _worker_chunks(c, s)
        _copy_acc(lambda n, r: zeros_hbm.at[pl.ds(0, n)],
                  lambda n, r: acc_sh.at[pl.ds(r, n)], s)
        _stage_idx(e_hbm, 1, wid, base, dst_v)
        pltpu.sync_copy(ones_hbm, ones_v)
        plsc.subcore_barrier()

        def body(j, carry):
            q = j // _SC4
            r = j - q * _SC4
            pltpu.sync_copy(ones_v, acc_sh.at[dst_v.at[q, r]], add=True)
            return carry

        lax.fori_loop(0, nj, body, 0)
        plsc.subcore_barrier()
        _copy_acc(lambda n, r: acc_sh.at[pl.ds(r, n)],
                  lambda n, r: out_hbm.at[c, pl.ds(r, n)], s)

    return deg_kernel


@functools.lru_cache(maxsize=None)
def _build_agg_kernel():
    mesh = plsc.VectorSubcoreMesh(core_axis_name="c", subcore_axis_name="s")

    @functools.partial(
        pl.kernel,
        mesh=mesh,
        out_type=jax.ShapeDtypeStruct((_NC, _N, _D), jnp.float32),
        scratch_types=[
            pltpu.VMEM((_SPW + 1, _SC4, _CHUNK), jnp.int32),
            pltpu.VMEM((_SPW + 1, _SC4, _CHUNK), jnp.int32),
            pltpu.VMEM((_HC, _D), jnp.float32),
            pltpu.VMEM((_HC, _D), jnp.float32),
            pltpu.VMEM_SHARED((_N, _D), jnp.float32),
            pltpu.SemaphoreType.DMA,
            pltpu.SemaphoreType.DMA,
        ],
    )
    def agg_kernel(h_hbm, e_hbm, zeros_hbm, out_hbm,
                   src_v, dst_v, rows_a, rows_b, acc_sh, sem_a, sem_b):
        c = lax.axis_index("c")
        s = lax.axis_index("s")
        wid, base, nj = _worker_chunks(c, s)
        _copy_acc(lambda n, r: zeros_hbm.at[pl.ds(0, n)],
                  lambda n, r: acc_sh.at[pl.ds(r, n)], s)
        _stage_idx(e_hbm, 0, wid, base, src_v)
        _stage_idx(e_hbm, 1, wid, base, dst_v)
        plsc.subcore_barrier()

        pltpu.async_copy(h_hbm.at[src_v.at[0, 0, pl.ds(0, _HC)]],
                         rows_a, sem_a)

        def body(i, carry):
            q = i // _SC4
            r = i - q * _SC4
            pltpu.async_copy(h_hbm.at[src_v.at[q, r, pl.ds(_HC, _HC)]],
                             rows_b, sem_b)
            pltpu.make_async_copy(h_hbm.at[src_v.at[q, r, pl.ds(0, _HC)]],
                                  rows_a, sem_a).wait()
            pltpu.sync_copy(rows_a, acc_sh.at[dst_v.at[q, r, pl.ds(0, _HC)]],
                            add=True)

            @pl.when(i < nj - 1)
            def _():
                q1 = (i + 1) // _SC4
                r1 = (i + 1) - q1 * _SC4
                pltpu.async_copy(h_hbm.at[src_v.at[q1, r1, pl.ds(0, _HC)]],
                                 rows_a, sem_a)

            pltpu.make_async_copy(h_hbm.at[src_v.at[q, r, pl.ds(_HC, _HC)]],
                                  rows_b, sem_b).wait()
            pltpu.sync_copy(rows_b, acc_sh.at[dst_v.at[q, r, pl.ds(_HC, _HC)]],
                            add=True)
            return carry

        lax.fori_loop(0, nj, body, 0)
        plsc.subcore_barrier()
        _copy_acc(lambda n, r: acc_sh.at[pl.ds(r, n)],
                  lambda n, r: out_hbm.at[c, pl.ds(r, n)], s)

    return agg_kernel


def _deg_call(e_r, zeros_rows, ones_deg):
    return _build_deg_kernel()(e_r, zeros_rows, ones_deg)


def _agg_call(hp, e_r, zeros_rows):
    return _build_agg_kernel()(hp, e_r, zeros_rows)


def _dinv_col(deg_ref):
    d = deg_ref[0, :, 0:1] + deg_ref[1, :, 0:1] + 1.0
    return lax.rsqrt(d)


def _matmul_t(a, w_ref):
    return lax.dot_general(a, w_ref[...], (((1,), (1,)), ((), ())),
                           preferred_element_type=jnp.float32,
                           precision=lax.Precision.HIGHEST)


def _ln_gelu(t, g_ref, be_ref):
    m = jnp.mean(t, axis=-1, keepdims=True)
    tc = t - m
    v = jnp.mean(tc * tc, axis=-1, keepdims=True)
    ln = tc * lax.rsqrt(v + 1e-5) * g_ref[...] + be_ref[...]
    return ln * 0.5 * (1.0 + lax.erf(ln * (2.0 ** -0.5)))


def _tc_in_body(x_ref, w_ref, deg_ref, o_ref):
    o_ref[...] = _dinv_col(deg_ref) * _matmul_t(x_ref[...], w_ref)


def _tc_mid_body(agg_ref, hp_ref, deg_ref, b_ref, g_ref, be_ref, w_ref, o_ref):
    dinv = _dinv_col(deg_ref)
    t = dinv * (agg_ref[0] + agg_ref[1] + hp_ref[...]) + b_ref[...]
    act = _ln_gelu(t, g_ref, be_ref)
    o_ref[...] = dinv * _matmul_t(act, w_ref)


def _tc_out_body(agg_ref, hp_ref, deg_ref, b_ref, g_ref, be_ref,
                 wh_ref, bh_ref, o_ref):
    dinv = _dinv_col(deg_ref)
    t = dinv * (agg_ref[0] + agg_ref[1] + hp_ref[...]) + b_ref[...]
    act = _ln_gelu(t, g_ref, be_ref)
    o_ref[...] = jnp.sum(act * wh_ref[...], axis=-1, keepdims=True) + bh_ref[...]


def _tc_in(x, W1, deg):
    return pl.pallas_call(
        _tc_in_body,
        out_shape=jax.ShapeDtypeStruct((_N, _D), jnp.float32),
    )(x, W1, deg)


def _tc_mid(agg, hp, deg, b, g, be, Wn):
    return pl.pallas_call(
        _tc_mid_body,
        out_shape=jax.ShapeDtypeStruct((_N, _D), jnp.float32),
    )(agg, hp, deg, b.reshape(1, _D), g.reshape(1, _D), be.reshape(1, _D), Wn)


def _tc_out(agg, hp, deg, b, g, be, Wh, bh):
    return pl.pallas_call(
        _tc_out_body,
        out_shape=jax.ShapeDtypeStruct((_N, 1), jnp.float32),
    )(agg, hp, deg, b.reshape(1, _D), g.reshape(1, _D), be.reshape(1, _D),
      Wh, bh.reshape(1, 1))


def kernel(x, edge_index, W1, b1, g1, be1, W2, b2, g2, be2, W3, b3, g3, be3,
           Wh, bh):
    e_r = edge_index.reshape(2, _ESC, _SC4, _CHUNK)
    zeros_rows = jnp.zeros((_RA, _D), jnp.float32)
    ones_deg = jnp.zeros((_CHUNK, _D), jnp.float32).at[:, 0].set(1.0)

    deg = _deg_call(e_r, zeros_rows, ones_deg)
    hp1 = _tc_in(x, W1, deg)
    agg1 = _agg_call(hp1, e_r, zeros_rows)
    hp2 = _tc_mid(agg1, hp1, deg, b1, g1, be1, W2)
    agg2 = _agg_call(hp2, e_r, zeros_rows)
    hp3 = _tc_mid(agg2, hp2, deg, b2, g2, be2, W3)
    agg3 = _agg_call(hp3, e_r, zeros_rows)
    y = _tc_out(agg3, hp3, deg, b3, g3, be3, Wh, bh)
    return y[:, 0]


# speedup vs baseline: 21.7544x; 1.0130x over previous
import functools

import jax
import jax.numpy as jnp
from jax import lax
from jax.experimental import pallas as pl
from jax.experimental.pallas import tpu as pltpu
from jax.experimental.pallas import tpu_sc as plsc

_N = 10000
_E = 320000
_D = 128
_NC = 2
_NS = 16
_NW = _NC * _NS
_CHUNK = 128
_HC = 64
_SC4 = 4
_ESC = _E // (_CHUNK * _SC4)
_SPW = _ESC // _NW
_XS = _ESC - _SPW * _NW
_RA = 632
_RB = _N - (_NS - 1) * _RA


def _worker_chunks(c, s):
    wid = s * _NC + c
    base = wid * _SPW + jnp.minimum(wid, _XS)
    nj = (_SPW + jnp.where(wid < _XS, 1, 0)) * _SC4
    return wid, base, nj


def _stage_idx(e_hbm, plane, wid, base, idx_v):
    @pl.when(wid < _XS)
    def _():
        pltpu.sync_copy(e_hbm.at[plane, pl.ds(base, _SPW + 1)], idx_v)

    @pl.when(wid >= _XS)
    def _():
        pltpu.sync_copy(e_hbm.at[plane, pl.ds(base, _SPW)],
                        idx_v.at[pl.ds(0, _SPW)])


def _copy_acc(src_fn, dst_fn, s):
    @pl.when(s < _NS - 1)
    def _():
        pltpu.sync_copy(src_fn(_RA, s * _RA), dst_fn(_RA, s * _RA))

    @pl.when(s == _NS - 1)
    def _():
        pltpu.sync_copy(src_fn(_RB, s * _RA), dst_fn(_RB, s * _RA))


@functools.lru_cache(maxsize=None)
def _build_deg_kernel():
    mesh = plsc.VectorSubcoreMesh(core_axis_name="c", subcore_axis_name="s")

    @functools.partial(
        pl.kernel,
        mesh=mesh,
        out_type=jax.ShapeDtypeStruct((_NC, _N, _D), jnp.float32),
        scratch_types=[
            pltpu.VMEM((_SPW + 1, _SC4, _CHUNK), jnp.int32),
            pltpu.VMEM((_CHUNK, _D), jnp.float32),
            pltpu.VMEM_SHARED((_N, _D), jnp.float32),
        ],
    )
    def deg_kernel(e_hbm, zeros_hbm, ones_hbm, out_hbm, dst_v, ones_v,
                   acc_sh):
        c = lax.axis_index("c")
        s = lax.axis_index("s")
        wid, base, nj = _worker_chunks(c, s)
        _copy_acc(lambda n, r: zeros_hbm.at[pl.ds(0, n)],
                  lambda n, r: acc_sh.at[pl.ds(r, n)], s)
        _stage_idx(e_hbm, 1, wid, base, dst_v)
        pltpu.sync_copy(ones_hbm, ones_v)
        plsc.subcore_barrier()

        def body(j, carry):
            q = j // _SC4
            r = j - q * _SC4
            pltpu.sync_copy(ones_v, acc_sh.at[dst_v.at[q, r]], add=True)
            return carry

        lax.fori_loop(0, nj, body, 0)
        plsc.subcore_barrier()
        _copy_acc(lambda n, r: acc_sh.at[pl.ds(r, n)],
                  lambda n, r: out_hbm.at[c, pl.ds(r, n)], s)

    return deg_kernel


@functools.lru_cache(maxsize=None)
def _build_agg_kernel():
    mesh = plsc.VectorSubcoreMesh(core_axis_name="c", subcore_axis_name="s")

    @functools.partial(
        pl.kernel,
        mesh=mesh,
        out_type=jax.ShapeDtypeStruct((_NC, _N, _D), jnp.float32),
        scratch_types=[
            pltpu.VMEM((_SPW + 1, _SC4, _CHUNK), jnp.int32),
            pltpu.VMEM((_SPW + 1, _SC4, _CHUNK), jnp.int32),
            pltpu.VMEM((_HC, _D), jnp.float32),
            pltpu.VMEM((_HC, _D), jnp.float32),
            pltpu.VMEM_SHARED((_N, _D), jnp.float32),
            pltpu.SemaphoreType.DMA,
            pltpu.SemaphoreType.DMA,
        ],
    )
    def agg_kernel(h_hbm, e_hbm, zeros_hbm, out_hbm,
                   src_v, dst_v, rows_a, rows_b, acc_sh, sem_a, sem_b):
        c = lax.axis_index("c")
        s = lax.axis_index("s")
        wid, base, nj = _worker_chunks(c, s)
        _copy_acc(lambda n, r: zeros_hbm.at[pl.ds(0, n)],
                  lambda n, r: acc_sh.at[pl.ds(r, n)], s)
        _stage_idx(e_hbm, 0, wid, base, src_v)
        _stage_idx(e_hbm, 1, wid, base, dst_v)
        plsc.subcore_barrier()

        pltpu.async_copy(h_hbm.at[src_v.at[0, 0, pl.ds(0, _HC)]],
                         rows_a, sem_a)

        def body(i, carry):
            q = i // _SC4
            r = i - q * _SC4
            pltpu.async_copy(h_hbm.at[src_v.at[q, r, pl.ds(_HC, _HC)]],
                             rows_b, sem_b)
            pltpu.make_async_copy(h_hbm.at[src_v.at[q, r, pl.ds(0, _HC)]],
                                  rows_a, sem_a).wait()
            pltpu.sync_copy(rows_a, acc_sh.at[dst_v.at[q, r, pl.ds(0, _HC)]],
                            add=True)

            @pl.when(i < nj - 1)
            def _():
                q1 = (i + 1) // _SC4
                r1 = (i + 1) - q1 * _SC4
                pltpu.async_copy(h_hbm.at[src_v.at[q1, r1, pl.ds(0, _HC)]],
                                 rows_a, sem_a)

            pltpu.make_async_copy(h_hbm.at[src_v.at[q, r, pl.ds(_HC, _HC)]],
                                  rows_b, sem_b).wait()
            pltpu.sync_copy(rows_b, acc_sh.at[dst_v.at[q, r, pl.ds(_HC, _HC)]],
                            add=True)
            return carry

        lax.fori_loop(0, nj, body, 0)
        plsc.subcore_barrier()
        _copy_acc(lambda n, r: acc_sh.at[pl.ds(r, n)],
                  lambda n, r: out_hbm.at[c, pl.ds(r, n)], s)

    return agg_kernel


def _deg_call(e_r, zeros_rows, ones_deg):
    return _build_deg_kernel()(e_r, zeros_rows, ones_deg)


def _agg_call(hp, e_r, zeros_rows):
    return _build_agg_kernel()(hp, e_r, zeros_rows)


def _dinv_col(deg_ref):
    d = deg_ref[0, :, 0:1] + deg_ref[1, :, 0:1] + 1.0
    return lax.rsqrt(d)


def _matmul_t(a, w_ref):
    return lax.dot_general(a, w_ref[...], (((1,), (1,)), ((), ())),
                           preferred_element_type=jnp.float32)


def _ln_gelu(t, g_ref, be_ref):
    m = jnp.mean(t, axis=-1, keepdims=True)
    tc = t - m
    v = jnp.mean(tc * tc, axis=-1, keepdims=True)
    ln = tc * lax.rsqrt(v + 1e-5) * g_ref[...] + be_ref[...]
    return ln * 0.5 * (1.0 + lax.erf(ln * (2.0 ** -0.5)))


def _tc_xw_body(x_ref, w_ref, o_ref):
    o_ref[...] = _matmul_t(x_ref[...], w_ref)


def _tc_scale_body(xw_ref, deg_ref, o_ref):
    o_ref[...] = _dinv_col(deg_ref) * xw_ref[...]


def _tc_mid_body(agg_ref, hp_ref, deg_ref, b_ref, g_ref, be_ref, w_ref, o_ref):
    dinv = _dinv_col(deg_ref)
    t = dinv * (agg_ref[0] + agg_ref[1] + hp_ref[...]) + b_ref[...]
    act = _ln_gelu(t, g_ref, be_ref)
    o_ref[...] = dinv * _matmul_t(act, w_ref)


def _tc_out_body(agg_ref, hp_ref, deg_ref, b_ref, g_ref, be_ref,
                 wh_ref, bh_ref, o_ref):
    dinv = _dinv_col(deg_ref)
    t = dinv * (agg_ref[0] + agg_ref[1] + hp_ref[...]) + b_ref[...]
    act = _ln_gelu(t, g_ref, be_ref)
    o_ref[...] = jnp.sum(act * wh_ref[...], axis=-1, keepdims=True) + bh_ref[...]


def _tc_in(x, W1, deg):
    xw = pl.pallas_call(
        _tc_xw_body,
        out_shape=jax.ShapeDtypeStruct((_N, _D), jnp.float32),
    )(x, W1)
    return pl.pallas_call(
        _tc_scale_body,
        out_shape=jax.ShapeDtypeStruct((_N, _D), jnp.float32),
    )(xw, deg)


def _tc_mid(agg, hp, deg, b, g, be, Wn):
    return pl.pallas_call(
        _tc_mid_body,
        out_shape=jax.ShapeDtypeStruct((_N, _D), jnp.float32),
    )(agg, hp, deg, b.reshape(1, _D), g.reshape(1, _D), be.reshape(1, _D), Wn)


def _tc_out(agg, hp, deg, b, g, be, Wh, bh):
    return pl.pallas_call(
        _tc_out_body,
        out_shape=jax.ShapeDtypeStruct((_N, 1), jnp.float32),
    )(agg, hp, deg, b.reshape(1, _D), g.reshape(1, _D), be.reshape(1, _D),
      Wh, bh.reshape(1, 1))


def kernel(x, edge_index, W1, b1, g1, be1, W2, b2, g2, be2, W3, b3, g3, be3,
           Wh, bh):
    e_r = edge_index.reshape(2, _ESC, _SC4, _CHUNK)
    zeros_rows = jnp.zeros((_RA, _D), jnp.float32)
    ones_deg = jnp.zeros((_CHUNK, _D), jnp.float32).at[:, 0].set(1.0)

    deg = _deg_call(e_r, zeros_rows, ones_deg)
    hp1 = _tc_in(x, W1, deg)
    agg1 = _agg_call(hp1, e_r, zeros_rows)
    hp2 = _tc_mid(agg1, hp1, deg, b1, g1, be1, W2)
    agg2 = _agg_call(hp2, e_r, zeros_rows)
    hp3 = _tc_mid(agg2, hp2, deg, b2, g2, be2, W3)
    agg3 = _agg_call(hp3, e_r, zeros_rows)
    y = _tc_out(agg3, hp3, deg, b3, g3, be3, Wh, bh)
    return y[:, 0]


# speedup vs baseline: 21.8645x; 1.0051x over previous
import functools

import jax
import jax.numpy as jnp
from jax import lax
from jax.experimental import pallas as pl
from jax.experimental.pallas import tpu as pltpu
from jax.experimental.pallas import tpu_sc as plsc

_N = 10000
_E = 320000
_D = 128
_NC = 2
_NS = 16
_NW = _NC * _NS
_CHUNK = 128
_HC = 64
_SC4 = 4
_ESC = _E // (_CHUNK * _SC4)
_SPW = _ESC // _NW
_XS = _ESC - _SPW * _NW
_RA = 632
_RB = _N - (_NS - 1) * _RA


def _worker_chunks(c, s):
    wid = s * _NC + c
    base = wid * _SPW + jnp.minimum(wid, _XS)
    nj = (_SPW + jnp.where(wid < _XS, 1, 0)) * _SC4
    return wid, base, nj


def _stage_idx(e_hbm, plane, wid, base, idx_v):
    @pl.when(wid < _XS)
    def _():
        pltpu.sync_copy(e_hbm.at[plane, pl.ds(base, _SPW + 1)], idx_v)

    @pl.when(wid >= _XS)
    def _():
        pltpu.sync_copy(e_hbm.at[plane, pl.ds(base, _SPW)],
                        idx_v.at[pl.ds(0, _SPW)])


def _copy_acc(src_fn, dst_fn, s):
    @pl.when(s < _NS - 1)
    def _():
        pltpu.sync_copy(src_fn(_RA, s * _RA), dst_fn(_RA, s * _RA))

    @pl.when(s == _NS - 1)
    def _():
        pltpu.sync_copy(src_fn(_RB, s * _RA), dst_fn(_RB, s * _RA))


@functools.lru_cache(maxsize=None)
def _build_deg_kernel():
    mesh = plsc.VectorSubcoreMesh(core_axis_name="c", subcore_axis_name="s")

    @functools.partial(
        pl.kernel,
        mesh=mesh,
        out_type=jax.ShapeDtypeStruct((_NC, _N, _D), jnp.float32),
        scratch_types=[
            pltpu.VMEM((_SPW + 1, _SC4, _CHUNK), jnp.int32),
            pltpu.VMEM((_CHUNK, _D), jnp.float32),
            pltpu.VMEM_SHARED((_N, _D), jnp.float32),
        ],
    )
    def deg_kernel(e_hbm, zeros_hbm, ones_hbm, out_hbm, dst_v, ones_v,
                   acc_sh):
        c = lax.axis_index("c")
        s = lax.axis_index("s")
        wid, base, nj = _worker_chunks(c, s)
        _copy_acc(lambda n, r: zeros_hbm.at[pl.ds(0, n)],
                  lambda n, r: acc_sh.at[pl.ds(r, n)], s)
        _stage_idx(e_hbm, 1, wid, base, dst_v)
        pltpu.sync_copy(ones_hbm, ones_v)
        plsc.subcore_barrier()

        def body(j, carry):
            q = j // _SC4
            r = j - q * _SC4
            pltpu.sync_copy(ones_v, acc_sh.at[dst_v.at[q, r]], add=True)
            return carry

        lax.fori_loop(0, nj, body, 0)
        plsc.subcore_barrier()
        _copy_acc(lambda n, r: acc_sh.at[pl.ds(r, n)],
                  lambda n, r: out_hbm.at[c, pl.ds(r, n)], s)

    return deg_kernel


@functools.lru_cache(maxsize=None)
def _build_agg_kernel():
    mesh = plsc.VectorSubcoreMesh(core_axis_name="c", subcore_axis_name="s")

    @functools.partial(
        pl.kernel,
        mesh=mesh,
        out_type=jax.ShapeDtypeStruct((_NC, _N, _D), jnp.float32),
        scratch_types=[
            pltpu.VMEM((_SPW + 1, _SC4, _CHUNK), jnp.int32),
            pltpu.VMEM((_SPW + 1, _SC4, _CHUNK), jnp.int32),
            pltpu.VMEM((_HC, _D), jnp.float32),
            pltpu.VMEM((_HC, _D), jnp.float32),
            pltpu.VMEM_SHARED((_N, _D), jnp.float32),
            pltpu.SemaphoreType.DMA,
            pltpu.SemaphoreType.DMA,
        ],
    )
    def agg_kernel(h_hbm, e_hbm, zeros_hbm, out_hbm,
                   src_v, dst_v, rows_a, rows_b, acc_sh, sem_a, sem_b):
        c = lax.axis_index("c")
        s = lax.axis_index("s")
        wid, base, nj = _worker_chunks(c, s)
        _stage_idx(e_hbm, 0, wid, base, src_v)
        pltpu.async_copy(h_hbm.at[src_v.at[0, 0, pl.ds(0, _HC)]],
                         rows_a, sem_a)
        pltpu.async_copy(h_hbm.at[src_v.at[0, 0, pl.ds(_HC, _HC)]],
                         rows_b, sem_b)
        _copy_acc(lambda n, r: zeros_hbm.at[pl.ds(0, n)],
                  lambda n, r: acc_sh.at[pl.ds(r, n)], s)
        _stage_idx(e_hbm, 1, wid, base, dst_v)
        plsc.subcore_barrier()

        def body(i, carry):
            q = i // _SC4
            r = i - q * _SC4
            pltpu.make_async_copy(h_hbm.at[src_v.at[q, r, pl.ds(0, _HC)]],
                                  rows_a, sem_a).wait()
            pltpu.sync_copy(rows_a, acc_sh.at[dst_v.at[q, r, pl.ds(0, _HC)]],
                            add=True)

            @pl.when(i < nj - 1)
            def _():
                q1 = (i + 1) // _SC4
                r1 = (i + 1) - q1 * _SC4
                pltpu.async_copy(h_hbm.at[src_v.at[q1, r1, pl.ds(0, _HC)]],
                                 rows_a, sem_a)

            pltpu.make_async_copy(h_hbm.at[src_v.at[q, r, pl.ds(_HC, _HC)]],
                                  rows_b, sem_b).wait()
            pltpu.sync_copy(rows_b, acc_sh.at[dst_v.at[q, r, pl.ds(_HC, _HC)]],
                            add=True)

            @pl.when(i < nj - 1)
            def _():
                q1 = (i + 1) // _SC4
                r1 = (i + 1) - q1 * _SC4
                pltpu.async_copy(h_hbm.at[src_v.at[q1, r1, pl.ds(_HC, _HC)]],
                                 rows_b, sem_b)

            return carry

        lax.fori_loop(0, nj, body, 0)
        plsc.subcore_barrier()
        _copy_acc(lambda n, r: acc_sh.at[pl.ds(r, n)],
                  lambda n, r: out_hbm.at[c, pl.ds(r, n)], s)

    return agg_kernel


def _deg_call(e_r, zeros_rows, ones_deg):
    return _build_deg_kernel()(e_r, zeros_rows, ones_deg)


def _agg_call(hp, e_r, zeros_rows):
    return _build_agg_kernel()(hp, e_r, zeros_rows)


def _dinv_col(deg_ref):
    d = deg_ref[0, :, 0:1] + deg_ref[1, :, 0:1] + 1.0
    return lax.rsqrt(d)


def _matmul_t(a, w_ref):
    return lax.dot_general(a, w_ref[...], (((1,), (1,)), ((), ())),
                           preferred_element_type=jnp.float32)


def _ln_gelu(t, g_ref, be_ref):
    m = jnp.mean(t, axis=-1, keepdims=True)
    tc = t - m
    v = jnp.mean(tc * tc, axis=-1, keepdims=True)
    ln = tc * lax.rsqrt(v + 1e-5) * g_ref[...] + be_ref[...]
    return ln * 0.5 * (1.0 + lax.erf(ln * (2.0 ** -0.5)))


def _tc_xw_body(x_ref, w_ref, o_ref):
    o_ref[...] = _matmul_t(x_ref[...], w_ref)


def _tc_scale_body(xw_ref, deg_ref, o_ref):
    o_ref[...] = _dinv_col(deg_ref) * xw_ref[...]


def _tc_mid_body(agg_ref, hp_ref, deg_ref, b_ref, g_ref, be_ref, w_ref, o_ref):
    dinv = _dinv_col(deg_ref)
    t = dinv * (agg_ref[0] + agg_ref[1] + hp_ref[...]) + b_ref[...]
    act = _ln_gelu(t, g_ref, be_ref)
    o_ref[...] = dinv * _matmul_t(act, w_ref)


def _tc_out_body(agg_ref, hp_ref, deg_ref, b_ref, g_ref, be_ref,
                 wh_ref, bh_ref, o_ref):
    dinv = _dinv_col(deg_ref)
    t = dinv * (agg_ref[0] + agg_ref[1] + hp_ref[...]) + b_ref[...]
    act = _ln_gelu(t, g_ref, be_ref)
    o_ref[...] = jnp.sum(act * wh_ref[...], axis=-1, keepdims=True) + bh_ref[...]


def _tc_in(x, W1, deg):
    xw = pl.pallas_call(
        _tc_xw_body,
        out_shape=jax.ShapeDtypeStruct((_N, _D), jnp.float32),
    )(x, W1)
    return pl.pallas_call(
        _tc_scale_body,
        out_shape=jax.ShapeDtypeStruct((_N, _D), jnp.float32),
    )(xw, deg)


def _tc_mid(agg, hp, deg, b, g, be, Wn):
    return pl.pallas_call(
        _tc_mid_body,
        out_shape=jax.ShapeDtypeStruct((_N, _D), jnp.float32),
    )(agg, hp, deg, b.reshape(1, _D), g.reshape(1, _D), be.reshape(1, _D), Wn)


def _tc_out(agg, hp, deg, b, g, be, Wh, bh):
    return pl.pallas_call(
        _tc_out_body,
        out_shape=jax.ShapeDtypeStruct((_N, 1), jnp.float32),
    )(agg, hp, deg, b.reshape(1, _D), g.reshape(1, _D), be.reshape(1, _D),
      Wh, bh.reshape(1, 1))


def kernel(x, edge_index, W1, b1, g1, be1, W2, b2, g2, be2, W3, b3, g3, be3,
           Wh, bh):
    e_r = edge_index.reshape(2, _ESC, _SC4, _CHUNK)
    zeros_rows = jnp.zeros((_RA, _D), jnp.float32)
    ones_deg = jnp.zeros((_CHUNK, _D), jnp.float32).at[:, 0].set(1.0)

    deg = _deg_call(e_r, zeros_rows, ones_deg)
    hp1 = _tc_in(x, W1, deg)
    agg1 = _agg_call(hp1, e_r, zeros_rows)
    hp2 = _tc_mid(agg1, hp1, deg, b1, g1, be1, W2)
    agg2 = _agg_call(hp2, e_r, zeros_rows)
    hp3 = _tc_mid(agg2, hp2, deg, b2, g2, be2, W3)
    agg3 = _agg_call(hp3, e_r, zeros_rows)
    y = _tc_out(agg3, hp3, deg, b3, g3, be3, Wh, bh)
    return y[:, 0]
